# baseline (speedup 1.0000x reference)
import functools
import math

import jax
import jax.numpy as jnp
from jax import lax
from jax.experimental import pallas as pl
from jax.experimental.pallas import tpu as pltpu
from jax.experimental.pallas import tpu_sc as plsc

NC, NS, LANES = 2, 16, 16
NW = NC * NS

R = 3
D = 128
BN = 4096
N1 = 10000
KR = 10240
ACC = 4224
GATHER_B = 32768


def _mesh():
    return plsc.VectorSubcoreMesh(core_axis_name="c", subcore_axis_name="s",
                                  num_cores=NC, num_subcores=NS)


def _sc_gather(table, idx):
    Bg = idx.shape[0]
    Dt = table.shape[1]
    bpw = Bg // NW
    chunk = 64
    nch = bpw // chunk

    @functools.partial(
        pl.kernel,
        out_type=jax.ShapeDtypeStruct((Bg, Dt), jnp.float32),
        mesh=_mesh(),
        scratch_types=[
            pltpu.VMEM((bpw,), jnp.int32),
            pltpu.VMEM((chunk, Dt), jnp.float32),
            pltpu.VMEM((chunk, Dt), jnp.float32),
            pltpu.SemaphoreType.DMA,
            pltpu.SemaphoreType.DMA,
        ],
    )
    def gk(table_hbm, idx_hbm, out_hbm, idx_v, buf0, buf1, sem0, sem1):
        wid = lax.axis_index("s") * NC + lax.axis_index("c")
        base = wid * bpw
        pltpu.sync_copy(idx_hbm.at[pl.ds(base, bpw)], idx_v)
        bufs = (buf0, buf1)
        sems = (sem0, sem1)
        cps = [None, None]
        for c in range(nch + 1):
            if c < nch:
                cps[c % 2] = pltpu.async_copy(
                    table_hbm.at[idx_v.at[pl.ds(c * chunk, chunk)]],
                    bufs[c % 2], sems[c % 2])
            if c >= 1:
                cps[(c - 1) % 2].wait()
                pltpu.sync_copy(bufs[(c - 1) % 2],
                                out_hbm.at[pl.ds(base + (c - 1) * chunk, chunk)])

    return gk(table, idx)


def _sc_edge(qtab, kvtab, src, dst, split_heads):
    E = src.shape[0] // R
    KT = kvtab.shape[0] // (R * (2 if split_heads else 1))
    QT = BN
    ACCk = BN if split_heads else BN // 2
    epw = E // NS
    C = 128
    SUPER = 2048
    nch = SUPER // C
    nsup = epw // SUPER
    rows_pt = ACCk // NS
    inv = 1.0 / math.sqrt(128.0)

    @functools.partial(
        pl.kernel,
        out_type=(jax.ShapeDtypeStruct((NC, R, ACCk, 128), jnp.float32),
                  jax.ShapeDtypeStruct((NW * R * ACCk,), jnp.float32)),
        mesh=_mesh(),
        scratch_types=[
            pltpu.VMEM((SUPER,), jnp.int32),
            pltpu.VMEM((SUPER,), jnp.int32),
            pltpu.VMEM((nch, C), jnp.int32),
            pltpu.VMEM((nch, C), jnp.float32),
            pltpu.VMEM((C, 128), jnp.float32),
            pltpu.VMEM((C, 256), jnp.float32),
            pltpu.VMEM((R * ACCk,), jnp.float32),
            pltpu.VMEM_SHARED((ACCk, 128), jnp.float32),
            pltpu.SemaphoreType.DMA,
            pltpu.SemaphoreType.DMA,
        ],
    )
    def ek(q_hbm, kv_hbm, src_hbm, dst_hbm, out_hbm, den_hbm,
           kvidxa, qidxa, scatall, validall, qrows, kvrows,
           dentile, acc, semq, semkv):
        cid = lax.axis_index("c")
        sid = lax.axis_index("s")
        wid = sid * NC + cid
        iot = lax.broadcasted_iota(jnp.int32, (16,), 0)
        zv = jnp.zeros((16,), jnp.float32)
        ebase = sid * epw

        def dzbody(i, _):
            dentile[pl.ds(i * 16, 16)] = zv
            return 0
        lax.fori_loop(0, R * ACCk // 16, dzbody, 0)

        def launch(c):
            pltpu.async_copy(
                q_hbm.at[qidxa.at[pl.ds(c * C, C)]], qrows, semq)
            pltpu.async_copy(
                kv_hbm.at[kvidxa.at[pl.ds(c * C, C)]], kvrows, semkv)

        def wait(c):
            pltpu.make_async_copy(
                q_hbm.at[qidxa.at[pl.ds(c * C, C)]], qrows, semq).wait()
            pltpu.make_async_copy(
                kv_hbm.at[kvidxa.at[pl.ds(c * C, C)]], kvrows, semkv).wait()

        def compute(c, r):
            wout = qrows

            def gbody(g, _):
                e0 = g * 16
                dm = scatall[c, pl.ds(e0, 16)]
                vf = validall[c, pl.ds(e0, 16)]
                for l in range(16):
                    e = e0 + l
                    a = zv
                    for j in range(8):
                        a = a + (qrows[e, pl.ds(j * 16, 16)]
                                 * kvrows[e, pl.ds(j * 16, 16)])
                    for m in (8, 4, 2, 1):
                        a = a + a.at[jnp.bitwise_xor(iot, m)].get(
                            mode="promise_in_bounds")
                    ex = jnp.exp(a * inv) * vf[l]
                    for j in range(8):
                        wout[e, pl.ds(j * 16, 16)] = (
                            kvrows[e, pl.ds(128 + j * 16, 16)] * ex)
                    idx = dm[l] + r * ACCk
                    dbase = (idx // 16) * 16
                    lane = idx - dbase
                    v = dentile[pl.ds(dbase, 16)]
                    dentile[pl.ds(dbase, 16)] = v + jnp.where(
                        iot == lane, ex, zv)
                return 0
            lax.fori_loop(0, C // 16, gbody, 0)
            pltpu.sync_copy(qrows, acc.at[scatall.at[c]], add=True)

        def rbody(r, _):
            if split_heads:
                qoff = (cid * R + r) * QT
                kvoff = (cid * R + r) * KT
            else:
                qoff = r * QT
                kvoff = r * KT

            def zbody(i, _):
                for j in range(8):
                    qrows[i, pl.ds(j * 16, 16)] = zv
                return 0
            lax.fori_loop(0, C, zbody, 0)
            rows0 = sid * rows_pt
            for off in range(0, rows_pt - C + 1, C):
                pltpu.sync_copy(qrows, acc.at[pl.ds(rows0 + off, C)])
            rem = rows_pt % C
            if rem:
                pltpu.sync_copy(qrows.at[pl.ds(0, rem)],
                                acc.at[pl.ds(rows0 + rows_pt - rem, rem)])
            plsc.subcore_barrier()

            def supbody(sup, _):
                rbase = r * E + ebase + sup * SUPER
                pltpu.sync_copy(src_hbm.at[pl.ds(rbase, SUPER)], kvidxa)
                pltpu.sync_copy(dst_hbm.at[pl.ds(rbase, SUPER)], qidxa)

                def ibody(ch, _):
                    for j in range(C // 16):
                        sl = pl.ds(ch * C + j * 16, 16)
                        jsl = pl.ds(j * 16, 16)
                        dv = qidxa[sl]
                        dc = jnp.minimum(dv, BN - 1)
                        if split_heads:
                            okb = dv < BN
                            scatall[ch, jsl] = dc
                        else:
                            okb = jnp.bitwise_and(dv, 1) == cid
                            scatall[ch, jsl] = dc >> 1
                        validall[ch, jsl] = jnp.where(
                            okb, jnp.ones((16,), jnp.float32), zv)
                        zi = jnp.zeros((16,), jnp.int32)
                        kvidxa[sl] = jnp.where(okb, kvidxa[sl], zi) + kvoff
                        qidxa[sl] = jnp.where(okb, dc, zi) + qoff
                    return 0
                lax.fori_loop(0, nch, ibody, 0)

                def chunkbody(c, _):
                    launch(c)
                    wait(c)
                    compute(c, r)
                    return 0
                lax.fori_loop(0, nch, chunkbody, 0)
                return 0
            lax.fori_loop(0, nsup, supbody, 0)

            plsc.subcore_barrier()
            pltpu.sync_copy(acc.at[pl.ds(sid * rows_pt, rows_pt)],
                            out_hbm.at[cid, r, pl.ds(sid * rows_pt, rows_pt)])
            plsc.subcore_barrier()
            return 0
        lax.fori_loop(0, R, rbody, 0)
        dlen = R * ACCk
        pltpu.sync_copy(dentile, den_hbm.at[pl.ds(wid * dlen, dlen)])

    return ek(qtab, kvtab, src, dst)


def _bmm(x, w, b, bm):
    Rr, M, K = x.shape
    N = w.shape[2]

    def body(x_ref, w_ref, b_ref, o_ref):
        o_ref[0] = (jnp.dot(x_ref[0], w_ref[0],
                            preferred_element_type=jnp.float32) + b_ref[0])

    return pl.pallas_call(
        body,
        grid=(Rr, M // bm),
        in_specs=[pl.BlockSpec((1, bm, K), lambda r, i: (r, i, 0)),
                  pl.BlockSpec((1, K, N), lambda r, i: (r, 0, 0)),
                  pl.BlockSpec((1, 1, N), lambda r, i: (r, 0, 0))],
        out_specs=pl.BlockSpec((1, bm, N), lambda r, i: (r, i, 0)),
        out_shape=jax.ShapeDtypeStruct((Rr, M, N), jnp.float32),
    )(x, w, b)


def _tc_mid(part, den, q0s, wba, wbb, gsc, bnb, w2, b2):
    BM = 512

    def body(p_ref, d_ref, qs_ref, wba_ref, wbb_ref, g_ref, bb_ref, w2_ref,
             b2_ref, o_ref):
        dn = jnp.sum(d_ref[0], axis=1)
        out = jnp.concatenate(
            [p_ref[0, 0] / (dn[:, 0:1] + 1e-16),
             p_ref[1, 0] / (dn[:, 1:2] + 1e-16)], axis=1)
        xr = qs_ref[0]
        g = jax.nn.sigmoid(jnp.sum(out * wba_ref[0], axis=1, keepdims=True)
                           + jnp.sum(xr * wbb_ref[0], axis=1, keepdims=True))
        y = g * xr + (1.0 - g) * out
        y = y * g_ref[0] + bb_ref[0]
        y = jnp.where(y > 0, y, jnp.exp(jnp.minimum(y, 0.0)) - 1.0)
        o_ref[0] = (jnp.dot(y, w2_ref[0], preferred_element_type=jnp.float32)
                    + b2_ref[0])

    return pl.pallas_call(
        body,
        grid=(R, BN // BM),
        in_specs=[
            pl.BlockSpec((2, 1, BM, 128), lambda r, i: (0, r, i, 0)),
            pl.BlockSpec((1, BM, 16, 2), lambda r, i: (r, i, 0, 0)),
            pl.BlockSpec((1, BM, 256), lambda r, i: (r, i, 1)),
            pl.BlockSpec((1, 1, 256), lambda r, i: (r, 0, 0)),
            pl.BlockSpec((1, 1, 256), lambda r, i: (r, 0, 0)),
            pl.BlockSpec((1, 1, 256), lambda r, i: (r, 0, 0)),
            pl.BlockSpec((1, 1, 256), lambda r, i: (r, 0, 0)),
            pl.BlockSpec((1, 256, 512), lambda r, i: (r, 0, 0)),
            pl.BlockSpec((1, 1, 512), lambda r, i: (r, 0, 0)),
        ],
        out_specs=pl.BlockSpec((1, BM, 512), lambda r, i: (r, i, 0)),
        out_shape=jax.ShapeDtypeStruct((R, BN, 512), jnp.float32),
    )(part, den, q0s, wba, wbb, gsc, bnb, w2, b2)


def _tc_fin(part, den, mid, wba, wbb):
    BM = 512

    def body(p_ref, d_ref, xr_ref, wba_ref, wbb_ref, o_ref):
        p = p_ref[0]
        dn = jnp.sum(d_ref[0], axis=1, keepdims=True)
        out = p / (dn + 1e-16)
        xr = xr_ref[0]
        g = jax.nn.sigmoid(jnp.sum(out * wba_ref[0], axis=1, keepdims=True)
                           + jnp.sum(xr * wbb_ref[0], axis=1, keepdims=True))
        o_ref[0] = g * xr + (1.0 - g) * out

    return pl.pallas_call(
        body,
        grid=(R, BN // BM),
        in_specs=[
            pl.BlockSpec((1, BM, 128), lambda r, i: (r, i, 0)),
            pl.BlockSpec((1, BM, 16), lambda r, i: (r, i, 0)),
            pl.BlockSpec((1, BM, 128), lambda r, i: (r, i, 3)),
            pl.BlockSpec((1, 1, 128), lambda r, i: (r, 0, 0)),
            pl.BlockSpec((1, 1, 128), lambda r, i: (r, 0, 0)),
        ],
        out_specs=pl.BlockSpec((1, BM, 128), lambda r, i: (r, i, 0)),
        out_shape=jax.ShapeDtypeStruct((R, BN, 128), jnp.float32),
    )(part, den, mid, wba, wbb)


def kernel(features, batch_nodes, n_ids, ei_l0, ei_l1, Wq1, bq1, Wk1, bk1,
           Wv1, bv1, Wskip1, bskip1, Wbeta1, bn_g, bn_b, Wq2, bq2, Wk2, bk2,
           Wv2, bv2, Wskip2, bskip2, Wbeta2):
    idxp = jnp.pad(n_ids[:, :N1], ((0, 0), (0, KR - N1)))
    idx_flat = jnp.pad(idxp.reshape(-1), (0, GATHER_B - R * KR))
    x0g = _sc_gather(features, idx_flat)
    x0p = x0g[:R * KR].reshape(R, KR, D)

    wkv1 = jnp.concatenate([Wk1, Wv1], axis=2)
    bkv1 = jnp.concatenate([bk1, bv1], axis=1).reshape(R, 1, 512)
    kv0 = _bmm(x0p, wkv1, bkv1, 1024)
    wqs1 = jnp.concatenate([Wq1, Wskip1], axis=2)
    bqs1 = jnp.concatenate([bq1, bskip1], axis=1).reshape(R, 1, 512)
    q0s = _bmm(x0p[:, :BN], wqs1, bqs1, 512)

    qh0 = (q0s[:, :, :256].reshape(R, BN, 2, 128)
           .transpose(2, 0, 1, 3).reshape(2 * R * BN, 128))
    t = kv0.reshape(R, KR, 4, 128)
    kvh0 = jnp.stack(
        [jnp.concatenate([t[:, :, 0], t[:, :, 2]], axis=-1),
         jnp.concatenate([t[:, :, 1], t[:, :, 3]], axis=-1)],
    ).reshape(2 * R * KR, 256)
    part0, den0 = _sc_edge(qh0, kvh0, ei_l0[:, 0, :].reshape(-1),
                           ei_l0[:, 1, :].reshape(-1), split_heads=True)
    den0t = den0.reshape(NS, NC, R, BN).transpose(2, 3, 0, 1)

    wb1 = Wbeta1[:, :, 0]
    wba1 = (wb1[:, 0:256] + wb1[:, 512:768]).reshape(R, 1, 256)
    wbb1 = (wb1[:, 256:512] - wb1[:, 512:768]).reshape(R, 1, 256)
    gsc = (bn_g / math.sqrt(1.0 + 1e-5)).reshape(R, 1, 256)
    bnb = bn_b.reshape(R, 1, 256)
    w2 = jnp.concatenate([Wq2, Wk2, Wv2, Wskip2], axis=2)
    b2 = jnp.concatenate([bq2, bk2, bv2, bskip2], axis=1).reshape(R, 1, 512)
    mid = _tc_mid(part0, den0t, q0s, wba1, wbb1, gsc, bnb, w2, b2)

    part1r, den1 = _sc_edge(mid[:, :, 0:128].reshape(R * BN, 128),
                            mid[:, :, 128:384].reshape(R * BN, 256),
                            ei_l1[:, 0, :].reshape(-1),
                            ei_l1[:, 1, :].reshape(-1), split_heads=False)
    part1 = part1r.transpose(1, 2, 0, 3).reshape(R, BN, 128)
    den1t = (den1.reshape(NS, NC, R, BN // 2)
             .transpose(2, 3, 1, 0).reshape(R, BN, NS))

    wb2 = Wbeta2[:, :, 0]
    wba2 = (wb2[:, 0:128] + wb2[:, 256:384]).reshape(R, 1, 128)
    wbb2 = (wb2[:, 128:256] - wb2[:, 256:384]).reshape(R, 1, 128)
    fin = _tc_fin(part1, den1t, mid, wba2, wbb2)
    return jnp.transpose(fin, (1, 0, 2)).reshape(BN, R * 128)


# speedup vs baseline: 7.2671x; 7.2671x over previous
import functools
import math

import jax
import jax.numpy as jnp
from jax import lax
from jax.experimental import pallas as pl
from jax.experimental.pallas import tpu as pltpu
from jax.experimental.pallas import tpu_sc as plsc

NC, NS, LANES = 2, 16, 16
NW = NC * NS

R = 3
D = 128
BN = 4096
N1 = 10000
KR = 10240
ACC = 4224
GATHER_B = 32768


def _mesh():
    return plsc.VectorSubcoreMesh(core_axis_name="c", subcore_axis_name="s",
                                  num_cores=NC, num_subcores=NS)


def _sc_gather(table, idx):
    Bg = idx.shape[0]
    Dt = table.shape[1]
    bpw = Bg // NW
    chunk = 64
    nch = bpw // chunk

    @functools.partial(
        pl.kernel,
        out_type=jax.ShapeDtypeStruct((Bg, Dt), jnp.float32),
        mesh=_mesh(),
        scratch_types=[
            pltpu.VMEM((bpw,), jnp.int32),
            pltpu.VMEM((chunk, Dt), jnp.float32),
            pltpu.VMEM((chunk, Dt), jnp.float32),
            pltpu.SemaphoreType.DMA,
            pltpu.SemaphoreType.DMA,
        ],
    )
    def gk(table_hbm, idx_hbm, out_hbm, idx_v, buf0, buf1, sem0, sem1):
        wid = lax.axis_index("s") * NC + lax.axis_index("c")
        base = wid * bpw
        pltpu.sync_copy(idx_hbm.at[pl.ds(base, bpw)], idx_v)
        bufs = (buf0, buf1)
        sems = (sem0, sem1)
        cps = [None, None]
        for c in range(nch + 1):
            if c < nch:
                cps[c % 2] = pltpu.async_copy(
                    table_hbm.at[idx_v.at[pl.ds(c * chunk, chunk)]],
                    bufs[c % 2], sems[c % 2])
            if c >= 1:
                cps[(c - 1) % 2].wait()
                pltpu.sync_copy(bufs[(c - 1) % 2],
                                out_hbm.at[pl.ds(base + (c - 1) * chunk, chunk)])

    return gk(table, idx)


def _sc_edge(qtab, kvtab, src, dst, split_heads):
    E = src.shape[0] // R
    KT = kvtab.shape[0] // (R * (2 if split_heads else 1))
    QT = BN
    ACCk = BN if split_heads else BN // 2
    epw = E // NS
    C = 64
    SUPER = 2048
    nch = SUPER // C
    nsup = epw // SUPER
    rows_pt = ACCk // NS
    inv = 1.0 / math.sqrt(128.0)

    @functools.partial(
        pl.kernel,
        out_type=(jax.ShapeDtypeStruct((NC, R, ACCk, 128), jnp.float32),
                  jax.ShapeDtypeStruct((NW * R * ACCk,), jnp.float32)),
        mesh=_mesh(),
        scratch_types=[
            pltpu.VMEM((SUPER,), jnp.int32),
            pltpu.VMEM((SUPER,), jnp.int32),
            pltpu.VMEM((nch, C), jnp.int32),
            pltpu.VMEM((SUPER,), jnp.int32),
            pltpu.VMEM((C, 128), jnp.float32),
            pltpu.VMEM((C, 128), jnp.float32),
            pltpu.VMEM((C, 256), jnp.float32),
            pltpu.VMEM((C, 256), jnp.float32),
            pltpu.VMEM((C, 128), jnp.float32),
            pltpu.VMEM((R * ACCk,), jnp.float32),
            pltpu.VMEM_SHARED((ACCk, 128), jnp.float32),
            pltpu.SemaphoreType.DMA,
            pltpu.SemaphoreType.DMA,
            pltpu.SemaphoreType.DMA,
            pltpu.SemaphoreType.DMA,
        ],
    )
    def ek(q_hbm, kv_hbm, src_hbm, dst_hbm, out_hbm, den_hbm,
           kvidxa, qidxa, scatall, scatflat, qr0, qr1, kr0, kr1, wout,
           dentile, acc, semq0, semkv0, semq1, semkv1):
        cid = lax.axis_index("c")
        sid = lax.axis_index("s")
        wid = sid * NC + cid
        iot = lax.broadcasted_iota(jnp.int32, (16,), 0)
        zv = jnp.zeros((16,), jnp.float32)
        ebase = sid * epw
        qbufs = (qr0, qr1)
        kbufs = (kr0, kr1)
        sems = ((semq0, semkv0), (semq1, semkv1))

        def dzbody(i, _):
            dentile[pl.ds(i * 16, 16)] = zv
            return 0
        lax.fori_loop(0, R * ACCk // 16, dzbody, 0)

        def szbody(i, _):
            scatflat[pl.ds(i * 16, 16)] = jnp.zeros((16,), jnp.int32)
            return 0
        lax.fori_loop(0, SUPER // 16, szbody, 0)

        def launch(c, b):
            cq = pltpu.async_copy(
                q_hbm.at[qidxa.at[pl.ds(c * C, C)]], qbufs[b], sems[b][0])
            ckv = pltpu.async_copy(
                kv_hbm.at[kvidxa.at[pl.ds(c * C, C)]], kbufs[b], sems[b][1])
            return cq, ckv

        def wait(c, b):
            pltpu.make_async_copy(
                q_hbm.at[qidxa.at[pl.ds(c * C, C)]], qbufs[b],
                sems[b][0]).wait()
            pltpu.make_async_copy(
                kv_hbm.at[kvidxa.at[pl.ds(c * C, C)]], kbufs[b],
                sems[b][1]).wait()

        def compute(c, b, r, nvalid):
            qrows = qbufs[b]
            kvrows = kbufs[b]

            def gbody(g, _):
                e0 = g * 16
                dm = scatall[c, pl.ds(e0, 16)]
                vf = jnp.where(c * C + e0 + iot < nvalid,
                               jnp.ones((16,), jnp.float32), zv)
                for l in range(16):
                    e = e0 + l
                    a = zv
                    for j in range(8):
                        a = a + (qrows[e, pl.ds(j * 16, 16)]
                                 * kvrows[e, pl.ds(j * 16, 16)])
                    for m in (8, 4, 2, 1):
                        a = a + a.at[jnp.bitwise_xor(iot, m)].get(
                            mode="promise_in_bounds")
                    ex = jnp.exp(a * inv) * vf[l]
                    for j in range(8):
                        wout[e, pl.ds(j * 16, 16)] = (
                            kvrows[e, pl.ds(128 + j * 16, 16)] * ex)
                    idx = dm[l] + r * ACCk
                    dbase = (idx // 16) * 16
                    lane = idx - dbase
                    v = dentile[pl.ds(dbase, 16)]
                    dentile[pl.ds(dbase, 16)] = v + jnp.where(
                        iot == lane, ex, zv)
                return 0
            lax.fori_loop(0, C // 16, gbody, 0)
            pltpu.sync_copy(wout, acc.at[scatall.at[c]], add=True)

        def rbody(r, _):
            if split_heads:
                qoff = (cid * R + r) * QT
                kvoff = (cid * R + r) * KT
            else:
                qoff = r * QT
                kvoff = r * KT

            def zbody(i, _):
                for j in range(8):
                    wout[i, pl.ds(j * 16, 16)] = zv
                return 0
            lax.fori_loop(0, C, zbody, 0)
            rows0 = sid * rows_pt
            for off in range(0, rows_pt - C + 1, C):
                pltpu.sync_copy(wout, acc.at[pl.ds(rows0 + off, C)])
            rem = rows_pt % C
            if rem:
                pltpu.sync_copy(wout.at[pl.ds(0, rem)],
                                acc.at[pl.ds(rows0 + rows_pt - rem, rem)])
            plsc.subcore_barrier()

            def supbody(sup, _):
                rbase = r * E + ebase + sup * SUPER
                pltpu.sync_copy(src_hbm.at[pl.ds(rbase, SUPER)], kvidxa)
                pltpu.sync_copy(dst_hbm.at[pl.ds(rbase, SUPER)], qidxa)

                def cbody(g, ptr):
                    sl = pl.ds(g * 16, 16)
                    sv = kvidxa[sl]
                    dv = qidxa[sl]
                    dc = jnp.minimum(dv, BN - 1)
                    if split_heads:
                        ok = dv < BN
                        sc = dc
                    else:
                        ok = jnp.bitwise_and(dv, 1) == cid
                        sc = dc >> 1
                    kvi = sv + kvoff
                    qi = dc + qoff
                    oki = jnp.where(ok, jnp.ones((16,), jnp.int32),
                                    jnp.zeros((16,), jnp.int32))
                    for l in range(16):
                        @pl.when(oki[l] != 0)
                        def _(ptr=ptr, l=l):
                            dbase = (ptr // 16) * 16
                            lane = ptr - dbase
                            dsl = pl.ds(dbase, 16)
                            m = iot == lane
                            kvidxa[dsl] = jnp.where(
                                m, jnp.full((16,), kvi[l], jnp.int32),
                                kvidxa[dsl])
                            qidxa[dsl] = jnp.where(
                                m, jnp.full((16,), qi[l], jnp.int32),
                                qidxa[dsl])
                            scatflat[dsl] = jnp.where(
                                m, jnp.full((16,), sc[l], jnp.int32),
                                scatflat[dsl])
                        ptr = ptr + oki[l]
                    return ptr
                nvalid = lax.fori_loop(0, SUPER // 16, cbody, 0)
                nch_c = (nvalid + C - 1) // C

                def pbody(ch, _):
                    for j in range(C // 16):
                        scatall[ch, pl.ds(j * 16, 16)] = jnp.minimum(
                            scatflat[pl.ds(ch * C + j * 16, 16)], ACCk - 1)
                    return 0
                lax.fori_loop(0, nch_c, pbody, 0)

                @pl.when(nch_c > 0)
                def _():
                    launch(0, 0)

                def pairbody(p, _):
                    c0 = 2 * p

                    @pl.when(c0 + 1 < nch_c)
                    def _():
                        launch(c0 + 1, 1)
                    wait(c0, 0)
                    compute(c0, 0, r, nvalid)

                    @pl.when(c0 + 2 < nch_c)
                    def _():
                        launch(c0 + 2, 0)

                    @pl.when(c0 + 1 < nch_c)
                    def _():
                        wait(c0 + 1, 1)
                        compute(c0 + 1, 1, r, nvalid)
                    return 0
                lax.fori_loop(0, (nch_c + 1) // 2, pairbody, 0)
                return 0
            lax.fori_loop(0, nsup, supbody, 0)

            plsc.subcore_barrier()
            pltpu.sync_copy(acc.at[pl.ds(sid * rows_pt, rows_pt)],
                            out_hbm.at[cid, r, pl.ds(sid * rows_pt, rows_pt)])
            plsc.subcore_barrier()
            return 0
        lax.fori_loop(0, R, rbody, 0)
        dlen = R * ACCk
        pltpu.sync_copy(dentile, den_hbm.at[pl.ds(wid * dlen, dlen)])

    return ek(qtab, kvtab, src, dst)


def _bmm(x, w, b, bm):
    Rr, M, K = x.shape
    N = w.shape[2]

    def body(x_ref, w_ref, b_ref, o_ref):
        o_ref[0] = (jnp.dot(x_ref[0], w_ref[0],
                            preferred_element_type=jnp.float32) + b_ref[0])

    return pl.pallas_call(
        body,
        grid=(Rr, M // bm),
        in_specs=[pl.BlockSpec((1, bm, K), lambda r, i: (r, i, 0)),
                  pl.BlockSpec((1, K, N), lambda r, i: (r, 0, 0)),
                  pl.BlockSpec((1, 1, N), lambda r, i: (r, 0, 0))],
        out_specs=pl.BlockSpec((1, bm, N), lambda r, i: (r, i, 0)),
        out_shape=jax.ShapeDtypeStruct((Rr, M, N), jnp.float32),
    )(x, w, b)


def _tc_mid(part, den, q0s, wba, wbb, gsc, bnb, w2, b2):
    BM = 512

    def body(p_ref, d_ref, qs_ref, wba_ref, wbb_ref, g_ref, bb_ref, w2_ref,
             b2_ref, o_ref):
        dn = jnp.sum(d_ref[0], axis=1)
        out = jnp.concatenate(
            [p_ref[0, 0] / (dn[:, 0:1] + 1e-16),
             p_ref[1, 0] / (dn[:, 1:2] + 1e-16)], axis=1)
        xr = qs_ref[0]
        g = jax.nn.sigmoid(jnp.sum(out * wba_ref[0], axis=1, keepdims=True)
                           + jnp.sum(xr * wbb_ref[0], axis=1, keepdims=True))
        y = g * xr + (1.0 - g) * out
        y = y * g_ref[0] + bb_ref[0]
        y = jnp.where(y > 0, y, jnp.exp(jnp.minimum(y, 0.0)) - 1.0)
        o_ref[0] = (jnp.dot(y, w2_ref[0], preferred_element_type=jnp.float32)
                    + b2_ref[0])

    return pl.pallas_call(
        body,
        grid=(R, BN // BM),
        in_specs=[
            pl.BlockSpec((2, 1, BM, 128), lambda r, i: (0, r, i, 0)),
            pl.BlockSpec((1, BM, 16, 2), lambda r, i: (r, i, 0, 0)),
            pl.BlockSpec((1, BM, 256), lambda r, i: (r, i, 1)),
            pl.BlockSpec((1, 1, 256), lambda r, i: (r, 0, 0)),
            pl.BlockSpec((1, 1, 256), lambda r, i: (r, 0, 0)),
            pl.BlockSpec((1, 1, 256), lambda r, i: (r, 0, 0)),
            pl.BlockSpec((1, 1, 256), lambda r, i: (r, 0, 0)),
            pl.BlockSpec((1, 256, 512), lambda r, i: (r, 0, 0)),
            pl.BlockSpec((1, 1, 512), lambda r, i: (r, 0, 0)),
        ],
        out_specs=pl.BlockSpec((1, BM, 512), lambda r, i: (r, i, 0)),
        out_shape=jax.ShapeDtypeStruct((R, BN, 512), jnp.float32),
    )(part, den, q0s, wba, wbb, gsc, bnb, w2, b2)


def _tc_fin(part, den, mid, wba, wbb):
    BM = 512

    def body(p_ref, d_ref, xr_ref, wba_ref, wbb_ref, o_ref):
        p = p_ref[0]
        dn = jnp.sum(d_ref[0], axis=1, keepdims=True)
        out = p / (dn + 1e-16)
        xr = xr_ref[0]
        g = jax.nn.sigmoid(jnp.sum(out * wba_ref[0], axis=1, keepdims=True)
                           + jnp.sum(xr * wbb_ref[0], axis=1, keepdims=True))
        o_ref[0] = g * xr + (1.0 - g) * out

    return pl.pallas_call(
        body,
        grid=(R, BN // BM),
        in_specs=[
            pl.BlockSpec((1, BM, 128), lambda r, i: (r, i, 0)),
            pl.BlockSpec((1, BM, 16), lambda r, i: (r, i, 0)),
            pl.BlockSpec((1, BM, 128), lambda r, i: (r, i, 3)),
            pl.BlockSpec((1, 1, 128), lambda r, i: (r, 0, 0)),
            pl.BlockSpec((1, 1, 128), lambda r, i: (r, 0, 0)),
        ],
        out_specs=pl.BlockSpec((1, BM, 128), lambda r, i: (r, i, 0)),
        out_shape=jax.ShapeDtypeStruct((R, BN, 128), jnp.float32),
    )(part, den, mid, wba, wbb)


def kernel(features, batch_nodes, n_ids, ei_l0, ei_l1, Wq1, bq1, Wk1, bk1,
           Wv1, bv1, Wskip1, bskip1, Wbeta1, bn_g, bn_b, Wq2, bq2, Wk2, bk2,
           Wv2, bv2, Wskip2, bskip2, Wbeta2):
    idxp = jnp.pad(n_ids[:, :N1], ((0, 0), (0, KR - N1)))
    idx_flat = jnp.pad(idxp.reshape(-1), (0, GATHER_B - R * KR))
    x0g = _sc_gather(features, idx_flat)
    x0p = x0g[:R * KR].reshape(R, KR, D)

    wkv1 = jnp.concatenate([Wk1, Wv1], axis=2)
    bkv1 = jnp.concatenate([bk1, bv1], axis=1).reshape(R, 1, 512)
    kv0 = _bmm(x0p, wkv1, bkv1, 1024)
    wqs1 = jnp.concatenate([Wq1, Wskip1], axis=2)
    bqs1 = jnp.concatenate([bq1, bskip1], axis=1).reshape(R, 1, 512)
    q0s = _bmm(x0p[:, :BN], wqs1, bqs1, 512)

    qh0 = (q0s[:, :, :256].reshape(R, BN, 2, 128)
           .transpose(2, 0, 1, 3).reshape(2 * R * BN, 128))
    t = kv0.reshape(R, KR, 4, 128)
    kvh0 = jnp.stack(
        [jnp.concatenate([t[:, :, 0], t[:, :, 2]], axis=-1),
         jnp.concatenate([t[:, :, 1], t[:, :, 3]], axis=-1)],
    ).reshape(2 * R * KR, 256)
    part0, den0 = _sc_edge(qh0, kvh0, ei_l0[:, 0, :].reshape(-1),
                           ei_l0[:, 1, :].reshape(-1), split_heads=True)
    den0t = den0.reshape(NS, NC, R, BN).transpose(2, 3, 0, 1)

    wb1 = Wbeta1[:, :, 0]
    wba1 = (wb1[:, 0:256] + wb1[:, 512:768]).reshape(R, 1, 256)
    wbb1 = (wb1[:, 256:512] - wb1[:, 512:768]).reshape(R, 1, 256)
    gsc = (bn_g / math.sqrt(1.0 + 1e-5)).reshape(R, 1, 256)
    bnb = bn_b.reshape(R, 1, 256)
    w2 = jnp.concatenate([Wq2, Wk2, Wv2, Wskip2], axis=2)
    b2 = jnp.concatenate([bq2, bk2, bv2, bskip2], axis=1).reshape(R, 1, 512)
    mid = _tc_mid(part0, den0t, q0s, wba1, wbb1, gsc, bnb, w2, b2)

    part1r, den1 = _sc_edge(mid[:, :, 0:128].reshape(R * BN, 128),
                            mid[:, :, 128:384].reshape(R * BN, 256),
                            ei_l1[:, 0, :].reshape(-1),
                            ei_l1[:, 1, :].reshape(-1), split_heads=False)
    part1 = part1r.transpose(1, 2, 0, 3).reshape(R, BN, 128)
    den1t = (den1.reshape(NS, NC, R, BN // 2)
             .transpose(2, 3, 1, 0).reshape(R, BN, NS))

    wb2 = Wbeta2[:, :, 0]
    wba2 = (wb2[:, 0:128] + wb2[:, 256:384]).reshape(R, 1, 128)
    wbb2 = (wb2[:, 128:256] - wb2[:, 256:384]).reshape(R, 1, 128)
    fin = _tc_fin(part1, den1t, mid, wba2, wbb2)
    return jnp.transpose(fin, (1, 0, 2)).reshape(BN, R * 128)
